# Initial kernel scaffold; baseline (speedup 1.0000x reference)
import jax
import jax.numpy as jnp
from jax.experimental import pallas as pl


def kernel(x, edge_index, batch, params):
    raise NotImplementedError("write your pallas kernel here")


# speedup vs baseline: 3.3107x; 3.3107x over previous
import jax
import jax.numpy as jnp
from jax import lax

HI = lax.Precision.HIGHEST
from jax.experimental import pallas as pl
from jax.experimental.pallas import tpu as pltpu
from jax.experimental.pallas import tpu_sc as plsc

F32 = jnp.float32

N = 10000
E = 320000
G = 16
W = 128
NPAD = 10240
NC, NS = 2, 16
NW = NC * NS
EW = E // NW
K = 80
CH = EW // K
NT_T = NPAD // NS


def _mesh():
    return plsc.VectorSubcoreMesh(core_axis_name="c", subcore_axis_name="s")


def _wid():
    return lax.axis_index("s") * NC + lax.axis_index("c")


def _sds(*shape):
    return jax.ShapeDtypeStruct(shape, F32)


def _sc_gather(dst, src, td, q):
    def body(dst_h, src_h, td_h, q_h, tdg_o, qsg_o, dsti, srci, tdv, qv, sem):
        base = _wid() * EW

        @pl.loop(0, CH)
        def _(j):
            off = base + j * K
            pltpu.sync_copy(dst_h.at[pl.ds(off, K)], dsti)
            pltpu.sync_copy(src_h.at[pl.ds(off, K)], srci)
            c1 = pltpu.async_copy(td_h.at[dsti], tdv, sem)
            c2 = pltpu.async_copy(q_h.at[srci], qv, sem)
            c1.wait()
            c2.wait()
            pltpu.sync_copy(tdv, tdg_o.at[pl.ds(off, K)])
            pltpu.sync_copy(qv, qsg_o.at[pl.ds(off, K)])

    return pl.kernel(
        body,
        out_type=(_sds(E, W), _sds(E, W)),
        mesh=_mesh(),
        scratch_types=(pltpu.VMEM((K,), jnp.int32),
                       pltpu.VMEM((K,), jnp.int32),
                       pltpu.VMEM((K, W), F32),
                       pltpu.VMEM((K, W), F32),
                       pltpu.SemaphoreType.DMA),
    )(dst, src, td, q)


def _sc_ssum(src, e):
    def body(src_h, e_h, out_h, sidx, ev, acc, sem):
        wid = _wid()
        lane = lax.iota(jnp.int32, 16)
        rowv = lax.rem(lane, 8)
        mlow = lane < 8
        mhigh = lane >= 8

        @pl.loop(0, NPAD // 16)
        def _(i):
            for r in range(8):
                acc[r, pl.ds(i * 16, 16)] = jnp.zeros((16,), F32)

        @pl.loop(0, CH)
        def _(j):
            off = wid * EW + j * K
            pltpu.sync_copy(src_h.at[pl.ds(off, K)], sidx)
            pltpu.sync_copy(e_h.at[pl.ds(off, K)], ev)

            @pl.loop(0, K // 16)
            def _(g):
                sl = pl.ds(g * 16, 16)
                iv = sidx[sl]
                xv = ev[sl]
                plsc.addupdate_scatter(acc, [rowv, iv], xv, mask=mlow)
                plsc.addupdate_scatter(acc, [rowv, iv], xv, mask=mhigh)

        @pl.loop(0, NPAD // 16)
        def _(i):
            sl = pl.ds(i * 16, 16)
            s = acc[0, sl]
            for r in range(1, 8):
                s = s + acc[r, sl]
            acc[0, sl] = s

        pltpu.sync_copy(acc.at[0], out_h.at[wid])

    return pl.kernel(
        body,
        out_type=_sds(NW, NPAD),
        mesh=_mesh(),
        scratch_types=(pltpu.VMEM((K,), jnp.int32),
                       pltpu.VMEM((K,), F32),
                       pltpu.VMEM((8, NPAD), F32),
                       pltpu.SemaphoreType.DMA),
        compiler_params=pltpu.CompilerParams(needs_layout_passes=False),
    )(src, e)


def _sc_coef(src, e, sinv):
    def body(src_h, e_h, s_h, cf_o, sidx, ev, cf, sbuf, sem):
        wid = _wid()
        pltpu.sync_copy(s_h, sbuf)

        @pl.loop(0, CH)
        def _(j):
            off = wid * EW + j * K
            pltpu.sync_copy(src_h.at[pl.ds(off, K)], sidx)
            pltpu.sync_copy(e_h.at[pl.ds(off, K)], ev)

            @pl.loop(0, K // 16)
            def _(g):
                sl = pl.ds(g * 16, 16)
                cf[sl] = ev[sl] * plsc.load_gather(sbuf, [sidx[sl]])

            pltpu.sync_copy(cf, cf_o.at[pl.ds(off, K)])

    return pl.kernel(
        body,
        out_type=_sds(E),
        mesh=_mesh(),
        scratch_types=(pltpu.VMEM((K,), jnp.int32),
                       pltpu.VMEM((K,), F32),
                       pltpu.VMEM((K,), F32),
                       pltpu.VMEM((NPAD,), F32),
                       pltpu.SemaphoreType.DMA),
        compiler_params=pltpu.CompilerParams(needs_layout_passes=False),
    )(src, e, sinv)


def _sc_agg(dst, wmsg):
    def body(dst_h, wm_h, out_h, idx2, wv, zv, aggs, sem):
        cid = lax.axis_index("c")
        sid = lax.axis_index("s")
        wid = sid * NC + cid

        @pl.loop(0, K)
        def _(r):
            @pl.loop(0, W // 16)
            def _(cc):
                zv[r, pl.ds(cc * 16, 16)] = jnp.zeros((16,), F32)

        @pl.loop(0, NT_T // K)
        def _(i):
            pltpu.sync_copy(zv, aggs.at[pl.ds(sid * NT_T + i * K, K)])

        plsc.subcore_barrier()

        @pl.loop(0, CH)
        def _(j):
            off = wid * EW + j * K
            pltpu.sync_copy(dst_h.at[pl.ds(off, K)], idx2.at[0])
            pltpu.sync_copy(wm_h.at[pl.ds(off, K)], wv)
            pltpu.sync_copy(wv, aggs.at[idx2.at[0]], add=True)

        plsc.subcore_barrier()

        @pl.loop(0, NT_T // K)
        def _(i):
            pltpu.sync_copy(aggs.at[pl.ds(sid * NT_T + i * K, K)],
                            out_h.at[cid, pl.ds(sid * NT_T + i * K, K)])

    return pl.kernel(
        body,
        out_type=_sds(NC, NPAD, W),
        mesh=_mesh(),
        scratch_types=(pltpu.VMEM((1, K), jnp.int32),
                       pltpu.VMEM((K, W), F32),
                       pltpu.VMEM((K, W), F32),
                       pltpu.VMEM_SHARED((NPAD, W), F32),
                       pltpu.SemaphoreType.DMA),
    )(dst, wmsg)


def _tc_node1(x, w1b, w2, b2):
    tn = 1000

    def body(h_r, w1b_r, w2_r, b2_r, q_o, xr_o):
        hb = h_r[...]
        q_o[...] = jnp.dot(hb, w1b_r[...], preferred_element_type=F32, precision=HI)
        xr_o[...] = jnp.maximum(
            jnp.dot(hb, w2_r[...], preferred_element_type=F32, precision=HI) + b2_r[...], 0.0)

    full = lambda s: pl.BlockSpec(s, lambda i: (0,) * len(s))
    return pl.pallas_call(
        body,
        grid=(N // tn,),
        in_specs=[pl.BlockSpec((tn, W), lambda i: (i, 0)),
                  full((W, W)), full((W, W)), full((1, W))],
        out_specs=[pl.BlockSpec((tn, W), lambda i: (i, 0)),
                   pl.BlockSpec((tn, W), lambda i: (i, 0))],
        out_shape=(_sds(N, W), _sds(N, W)),
    )(x, w1b, w2, b2)


def _tc_node2(h, w1d, b1, w5, b5, w7t, w1b, w2, b2):
    tn = 1000

    def body(h_r, w1d_r, b1_r, w5_r, b5_r, w7t_r, w1b_r, w2_r, b2_r,
             td_o, q_o, xr_o):
        hb = h_r[...]
        p = jnp.dot(hb, w1d_r[...], preferred_element_type=F32, precision=HI) + b1_r[...]
        v = jnp.tanh(jnp.dot(hb, w5_r[...], preferred_element_type=F32, precision=HI)
                     + b5_r[...])
        td_o[...] = p + jnp.dot(v, w7t_r[...], preferred_element_type=F32, precision=HI)
        q_o[...] = jnp.dot(hb, w1b_r[...], preferred_element_type=F32, precision=HI)
        xr_o[...] = jnp.maximum(
            jnp.dot(hb, w2_r[...], preferred_element_type=F32, precision=HI) + b2_r[...], 0.0)

    full = lambda s: pl.BlockSpec(s, lambda i: (0,) * len(s))
    return pl.pallas_call(
        body,
        grid=(N // tn,),
        in_specs=[pl.BlockSpec((tn, W), lambda i: (i, 0)),
                  full((W, W)), full((1, W)),
                  full((W, 16)), full((1, 16)), full((16, W)),
                  full((W, W)), full((W, W)), full((1, W))],
        out_specs=[pl.BlockSpec((tn, W), lambda i: (i, 0)),
                   pl.BlockSpec((tn, W), lambda i: (i, 0)),
                   pl.BlockSpec((tn, W), lambda i: (i, 0))],
        out_shape=(_sds(N, W), _sds(N, W), _sds(N, W)),
    )(h, w1d, b1, w5, b5, w7t, w1b, w2, b2)


def _tc_edge1(xd, qs, w1d, b1, w5, b5, w7, w6, b6):
    te = 1000

    def body(xd_r, qs_r, w1d_r, b1_r, w5_r, b5_r, w7_r, w6_r, b6_r,
             msg_o, e_o):
        xdb = xd_r[...]
        msg = jnp.maximum(
            jnp.dot(xdb, w1d_r[...], preferred_element_type=F32, precision=HI) + b1_r[...]
            + qs_r[...], 0.0)
        msg_o[...] = msg
        vd = jnp.tanh(jnp.dot(xdb, w5_r[...], preferred_element_type=F32, precision=HI)
                      + b5_r[...]) * w7_r[...]
        t2 = jnp.tanh(jnp.dot(msg, w6_r[...], preferred_element_type=F32, precision=HI)
                      + b6_r[...])
        e_o[...] = jnp.exp(jnp.sum(vd * t2, axis=1, keepdims=True))

    full = lambda s: pl.BlockSpec(s, lambda i: (0,) * len(s))
    return pl.pallas_call(
        body,
        grid=(E // te,),
        in_specs=[pl.BlockSpec((te, W), lambda i: (i, 0)),
                  pl.BlockSpec((te, W), lambda i: (i, 0)),
                  full((W, W)), full((1, W)),
                  full((W, 16)), full((1, 16)), full((1, 16)),
                  full((W, 16)), full((1, 16))],
        out_specs=[pl.BlockSpec((te, W), lambda i: (i, 0)),
                   pl.BlockSpec((te, 1), lambda i: (i, 0))],
        out_shape=(_sds(E, W), _sds(E, 1)),
    )(xd, qs, w1d, b1, w5, b5, w7, w6, b6)


def _tc_edge2(td, qs, selv, w6, b6):
    te = 1000

    def body(td_r, qs_r, selv_r, w6_r, b6_r, msg_o, e_o):
        tdb = td_r[...]
        msg = jnp.maximum(tdb + qs_r[...], 0.0)
        msg_o[...] = msg
        vd = jnp.dot(tdb, selv_r[...], preferred_element_type=F32, precision=HI)
        t2 = jnp.tanh(jnp.dot(msg, w6_r[...], preferred_element_type=F32, precision=HI)
                      + b6_r[...])
        e_o[...] = jnp.exp(jnp.sum(vd * t2, axis=1, keepdims=True))

    full = lambda s: pl.BlockSpec(s, lambda i: (0,) * len(s))
    return pl.pallas_call(
        body,
        grid=(E // te,),
        in_specs=[pl.BlockSpec((te, W), lambda i: (i, 0)),
                  pl.BlockSpec((te, W), lambda i: (i, 0)),
                  full((W, 16)), full((W, 16)), full((1, 16))],
        out_specs=[pl.BlockSpec((te, W), lambda i: (i, 0)),
                   pl.BlockSpec((te, 1), lambda i: (i, 0))],
        out_shape=(_sds(E, W), _sds(E, 1)),
    )(td, qs, selv, w6, b6)


def _tc_reduce(partials3):
    def body(p_r, o_r):
        s = jnp.sum(p_r[...], axis=0)
        o_r[...] = 1.0 / (s + 1e-16)

    return pl.pallas_call(
        body,
        grid=(1,),
        in_specs=[pl.BlockSpec((NW, 8, NPAD // 8), lambda i: (0, 0, 0))],
        out_specs=pl.BlockSpec((8, NPAD // 8), lambda i: (0, 0)),
        out_shape=_sds(8, NPAD // 8),
    )(partials3)


def _tc_scale(msg, coef2):
    te = 1000

    def body(m_r, c_r, o_r):
        o_r[...] = m_r[...] * c_r[...]

    return pl.pallas_call(
        body,
        grid=(E // te,),
        in_specs=[pl.BlockSpec((te, W), lambda i: (i, 0)),
                  pl.BlockSpec((te, 1), lambda i: (i, 0))],
        out_specs=pl.BlockSpec((te, W), lambda i: (i, 0)),
        out_shape=_sds(E, W),
    )(msg, coef2)


def _tc_epilogue(aggp, xr, w3x, w3a, b3, w4x, w4a, b4):
    tn = 1000

    def body(ag_r, xr_r, w3x_r, w3a_r, b3_r, w4x_r, w4a_r, b4_r, o_r):
        agg = ag_r[0] + ag_r[1]
        xr_b = xr_r[...]
        z3 = (jnp.sum(xr_b * w3x_r[...], axis=1, keepdims=True)
              + jnp.sum(agg * w3a_r[...], axis=1, keepdims=True) + b3_r[0, 0])
        z4 = (jnp.sum(xr_b * w4x_r[...], axis=1, keepdims=True)
              + jnp.sum(agg * w4a_r[...], axis=1, keepdims=True) + b4_r[0, 0])
        a1 = 1.0 / (1.0 + jnp.exp(-z3))
        a2 = 1.0 / (1.0 + jnp.exp(-z4))
        o_r[...] = jnp.maximum(a1 * agg + a2 * xr_b, 0.0)

    full = lambda s: pl.BlockSpec(s, lambda i: (0,) * len(s))
    return pl.pallas_call(
        body,
        grid=(N // tn,),
        in_specs=[pl.BlockSpec((NC, tn, W), lambda i: (0, i, 0)),
                  pl.BlockSpec((tn, W), lambda i: (i, 0)),
                  full((1, W)), full((1, W)), full((1, 1)),
                  full((1, W)), full((1, W)), full((1, 1))],
        out_specs=pl.BlockSpec((tn, W), lambda i: (i, 0)),
        out_shape=_sds(N, W),
    )(aggp, xr, w3x, w3a, b3, w4x, w4a, b4)


def _tc_pool(h, batch2, wgt, bg):
    def body(h_r, b_r, wg_r, bg_r, o_r):
        hb = h_r[...]
        gate = jnp.sum(hb * wg_r[...], axis=1, keepdims=True) + bg_r[0, 0]
        gids = lax.broadcasted_iota(jnp.int32, (1, G), 1)
        onehot = b_r[...] == gids
        mg = jnp.max(jnp.where(onehot, gate, -1e30), axis=0, keepdims=True)
        e2 = jnp.where(onehot, jnp.exp(gate - mg), 0.0)
        s = jnp.sum(e2, axis=0, keepdims=True)
        c2 = e2 / (s + 1e-16)
        o_r[...] = lax.dot_general(c2, hb, (((0,), (0,)), ((), ())),
                                   preferred_element_type=F32, precision=HI)

    return pl.pallas_call(
        body,
        grid=(1,),
        in_specs=[pl.BlockSpec((N, W), lambda i: (0, 0)),
                  pl.BlockSpec((N, 1), lambda i: (0, 0)),
                  pl.BlockSpec((1, W), lambda i: (0, 0)),
                  pl.BlockSpec((1, 1), lambda i: (0, 0))],
        out_specs=pl.BlockSpec((G, W), lambda i: (0, 0)),
        out_shape=_sds(G, W),
    )(h, batch2, wgt, bg)


def _tc_head(x1, x2, w1x, w1y, l1b, l2w, l2b, l3w, l3b, l4wt, l4b):
    def body(x1_r, x2_r, w1x_r, w1y_r, b1_r, w2_r, b2_r, w3_r, b3_r,
             w4_r, b4_r, o_r):
        h1 = jnp.maximum(
            jnp.dot(x1_r[...], w1x_r[...], preferred_element_type=F32, precision=HI)
            + jnp.dot(x2_r[...], w1y_r[...], preferred_element_type=F32, precision=HI)
            + b1_r[...], 0.0)
        h2 = jnp.maximum(jnp.dot(h1, w2_r[...], preferred_element_type=F32, precision=HI)
                         + b2_r[...], 0.0)
        h3 = jnp.dot(h2, w3_r[...], preferred_element_type=F32, precision=HI) + b3_r[...]
        o_r[...] = jnp.sum(h3 * w4_r[...], axis=1, keepdims=True) + b4_r[0, 0]

    full = lambda s: pl.BlockSpec(s, lambda i: (0,) * len(s))
    return pl.pallas_call(
        body,
        grid=(1,),
        in_specs=[full((G, W)), full((G, W)),
                  full((W, W)), full((W, W)), full((1, W)),
                  full((W, 16)), full((1, 16)),
                  full((16, 16)), full((1, 16)),
                  full((1, 16)), full((1, 1))],
        out_specs=full((G, 1)),
        out_shape=_sds(G, 1),
    )(x1, x2, w1x, w1y, l1b, l2w, l2b, l3w, l3b, l4wt, l4b)


def _pad2(w, rows, cols):
    return jnp.pad(w, ((0, rows - w.shape[0]), (0, cols - w.shape[1])))


def _pad1(b, n):
    return jnp.pad(b, (0, n - b.shape[0]))


def _conv1(x, dst, src, p):
    fin = fout = 128
    w1 = p["mlp1"]["w"]
    w1d = w1[:fin] - w1[fin:]
    q, xr = _tc_node1(x, w1[fin:], p["mlp2"]["w"], p["mlp2"]["b"][None, :])
    xd, qs = _sc_gather(dst, src, x, q)
    msg, e2 = _tc_edge1(xd, qs, w1d, p["mlp1"]["b"][None, :],
                        p["mlp5"]["w"], p["mlp5"]["b"][None, :],
                        p["mlp7"]["w"][:, 0][None, :],
                        p["mlp6"]["w"], p["mlp6"]["b"][None, :])
    return _edge_tail(dst, src, msg, e2, xr, p, fout)


def _conv2(h, dst, src, p, fin):
    fout = 64
    w1 = p["mlp1"]["w"]
    w1a, w1b = w1[:fin], w1[fin:]
    w1d_p = _pad2(w1a - w1b, W, W)
    w1b_p = _pad2(w1b, W, W)
    b1_p = _pad1(p["mlp1"]["b"], W)[None, :]
    w5_p = _pad2(p["mlp5"]["w"], W, 16)
    w7t = jnp.zeros((16, W), F32).at[
        jnp.arange(16), fout + jnp.arange(16)].set(p["mlp7"]["w"][:, 0])
    w2_p = _pad2(p["mlp2"]["w"], W, W)
    b2_p = _pad1(p["mlp2"]["b"], W)[None, :]
    td, q, xr = _tc_node2(h, w1d_p, b1_p, w5_p, p["mlp5"]["b"][None, :],
                          w7t, w1b_p, w2_p, b2_p)
    tdg, qsg = _sc_gather(dst, src, td, q)
    selv = jnp.zeros((W, 16), F32).at[
        fout + jnp.arange(16), jnp.arange(16)].set(1.0)
    w6_p = _pad2(p["mlp6"]["w"], W, 16)
    msg, e2 = _tc_edge2(tdg, qsg, selv, w6_p, p["mlp6"]["b"][None, :])
    return _edge_tail(dst, src, msg, e2, xr, p, fout)


def _edge_tail(dst, src, msg, e2, xr, p, fout):
    e1 = e2.reshape(E)
    partials = _sc_ssum(src, e1)
    sinv = _tc_reduce(partials.reshape(NW, 8, NPAD // 8)).reshape(NPAD)
    coef = _sc_coef(src, e1, sinv)
    wmsg = _tc_scale(msg, coef.reshape(E, 1))
    aggp = _sc_agg(dst, wmsg)
    w3 = p["mlp3"]["w"][:, 0]
    w4 = p["mlp4"]["w"][:, 0]
    return _tc_epilogue(aggp, xr,
                        _pad1(w3[:fout], W)[None, :],
                        _pad1(w3[fout:], W)[None, :],
                        p["mlp3"]["b"][None, None, 0],
                        _pad1(w4[:fout], W)[None, :],
                        _pad1(w4[fout:], W)[None, :],
                        p["mlp4"]["b"][None, None, 0])


def kernel(x, edge_index, batch, params):
    assert x.shape == (N, 128) and edge_index.shape == (2, E)
    src = edge_index[0]
    dst = edge_index[1]
    batch2 = batch.reshape(N, 1)

    h = _conv1(x, dst, src, params["conv1"])
    h = _conv2(h, dst, src, params["conv2"], 128)
    g1 = params["gate1"]
    x1 = _tc_pool(h, batch2, _pad1(g1["w"][:, 0], W)[None, :],
                  g1["b"][None, None, 0])
    h = _conv2(h, dst, src, params["conv3"], 64)
    h = _conv2(h, dst, src, params["conv4"], 64)
    g2 = params["gate2"]
    x2 = _tc_pool(h, batch2, _pad1(g2["w"][:, 0], W)[None, :],
                  g2["b"][None, None, 0])

    pr = params
    w1x = _pad2(pr["lin1"]["w"][:64], W, W)
    w1y = _pad2(pr["lin1"]["w"][64:], W, W)
    out = _tc_head(
        x1, x2, w1x, w1y, pr["lin1"]["b"][None, :],
        pr["lin2"]["w"], pr["lin2"]["b"][None, :],
        pr["lin3"]["w"], pr["lin3"]["b"][None, :],
        pr["lin4"]["w"][:, 0][None, :], pr["lin4"]["b"][None, None, 0])
    return out[:, 0]


# speedup vs baseline: 3.5302x; 1.0663x over previous
import jax
import jax.numpy as jnp
from jax import lax

HI = lax.Precision.HIGHEST
from jax.experimental import pallas as pl
from jax.experimental.pallas import tpu as pltpu
from jax.experimental.pallas import tpu_sc as plsc

F32 = jnp.float32

N = 10000
E = 320000
G = 16
W = 128
NPAD = 10240
NC, NS = 2, 16
NW = NC * NS
EW = E // NW
K = 80
CH = EW // K
NT_T = NPAD // NS


def _mesh():
    return plsc.VectorSubcoreMesh(core_axis_name="c", subcore_axis_name="s")


def _wid():
    return lax.axis_index("s") * NC + lax.axis_index("c")


def _sds(*shape):
    return jax.ShapeDtypeStruct(shape, F32)


def _sc_gather(dst, src, td, q):
    wt = td.shape[1]
    wq = q.shape[1]

    def body(dst_h, src_h, td_h, q_h, tdg_o, qsg_o, dsti, srci, tdv, qv,
             sem0, sem1):
        base = _wid() * EW

        def run(off, b, sem):
            pltpu.sync_copy(dst_h.at[pl.ds(off, K)], dsti.at[b])
            pltpu.sync_copy(src_h.at[pl.ds(off, K)], srci.at[b])
            c1 = pltpu.async_copy(td_h.at[dsti.at[b]], tdv.at[b], sem)
            c2 = pltpu.async_copy(q_h.at[srci.at[b]], qv.at[b], sem)
            return c1, c2

        def drain(off, b, c1, c2):
            c1.wait()
            c2.wait()
            pltpu.sync_copy(tdv.at[b], tdg_o.at[pl.ds(off, K)])
            pltpu.sync_copy(qv.at[b], qsg_o.at[pl.ds(off, K)])

        @pl.loop(0, CH // 2)
        def _(j):
            o0 = base + (2 * j) * K
            o1 = o0 + K
            c1, c2 = run(o0, 0, sem0)
            c3, c4 = run(o1, 1, sem1)
            drain(o0, 0, c1, c2)
            drain(o1, 1, c3, c4)

        if CH % 2:
            ot = base + (CH - 1) * K
            c1, c2 = run(ot, 0, sem0)
            drain(ot, 0, c1, c2)

    return pl.kernel(
        body,
        out_type=(_sds(E, wt), _sds(E, wq)),
        mesh=_mesh(),
        scratch_types=(pltpu.VMEM((2, K), jnp.int32),
                       pltpu.VMEM((2, K), jnp.int32),
                       pltpu.VMEM((2, K, wt), F32),
                       pltpu.VMEM((2, K, wq), F32),
                       pltpu.SemaphoreType.DMA,
                       pltpu.SemaphoreType.DMA),
    )(dst, src, td, q)


def _sc_ssum(src, e):
    def body(src_h, e_h, out_h, sidx, ev, acc, sem):
        wid = _wid()
        lane = lax.iota(jnp.int32, 16)
        rowv = lax.rem(lane, 8)
        mlow = lane < 8
        mhigh = lane >= 8

        @pl.loop(0, NPAD // 16)
        def _(i):
            for r in range(8):
                acc[r, pl.ds(i * 16, 16)] = jnp.zeros((16,), F32)

        @pl.loop(0, CH)
        def _(j):
            off = wid * EW + j * K
            pltpu.sync_copy(src_h.at[pl.ds(off, K)], sidx)
            pltpu.sync_copy(e_h.at[pl.ds(off, K)], ev)

            @pl.loop(0, K // 16)
            def _(g):
                sl = pl.ds(g * 16, 16)
                iv = sidx[sl]
                xv = ev[sl]
                plsc.addupdate_scatter(acc, [rowv, iv], xv, mask=mlow)
                plsc.addupdate_scatter(acc, [rowv, iv], xv, mask=mhigh)

        @pl.loop(0, NPAD // 16)
        def _(i):
            sl = pl.ds(i * 16, 16)
            s = acc[0, sl]
            for r in range(1, 8):
                s = s + acc[r, sl]
            acc[0, sl] = s

        pltpu.sync_copy(acc.at[0], out_h.at[wid])

    return pl.kernel(
        body,
        out_type=_sds(NW, NPAD),
        mesh=_mesh(),
        scratch_types=(pltpu.VMEM((K,), jnp.int32),
                       pltpu.VMEM((K,), F32),
                       pltpu.VMEM((8, NPAD), F32),
                       pltpu.SemaphoreType.DMA),
        compiler_params=pltpu.CompilerParams(needs_layout_passes=False),
    )(src, e)


def _sc_coef(src, e, sinv):
    def body(src_h, e_h, s_h, cf_o, sidx, ev, cf, sbuf, sem):
        wid = _wid()
        pltpu.sync_copy(s_h, sbuf)

        @pl.loop(0, CH)
        def _(j):
            off = wid * EW + j * K
            pltpu.sync_copy(src_h.at[pl.ds(off, K)], sidx)
            pltpu.sync_copy(e_h.at[pl.ds(off, K)], ev)

            @pl.loop(0, K // 16)
            def _(g):
                sl = pl.ds(g * 16, 16)
                cf[sl] = ev[sl] * plsc.load_gather(sbuf, [sidx[sl]])

            pltpu.sync_copy(cf, cf_o.at[pl.ds(off, K)])

    return pl.kernel(
        body,
        out_type=_sds(E),
        mesh=_mesh(),
        scratch_types=(pltpu.VMEM((K,), jnp.int32),
                       pltpu.VMEM((K,), F32),
                       pltpu.VMEM((K,), F32),
                       pltpu.VMEM((NPAD,), F32),
                       pltpu.SemaphoreType.DMA),
        compiler_params=pltpu.CompilerParams(needs_layout_passes=False),
    )(src, e, sinv)


def _sc_agg(dst, wmsg):
    wm = wmsg.shape[1]

    def body(dst_h, wm_h, out_h, idx2, wv, zv, aggs, sem):
        cid = lax.axis_index("c")
        sid = lax.axis_index("s")
        wid = sid * NC + cid

        @pl.loop(0, K)
        def _(r):
            @pl.loop(0, wm // 16)
            def _(cc):
                zv[r, pl.ds(cc * 16, 16)] = jnp.zeros((16,), F32)

        @pl.loop(0, NT_T // K)
        def _(i):
            pltpu.sync_copy(zv, aggs.at[pl.ds(sid * NT_T + i * K, K)])

        plsc.subcore_barrier()

        @pl.loop(0, CH)
        def _(j):
            off = wid * EW + j * K
            pltpu.sync_copy(dst_h.at[pl.ds(off, K)], idx2.at[0])
            pltpu.sync_copy(wm_h.at[pl.ds(off, K)], wv)
            pltpu.sync_copy(wv, aggs.at[idx2.at[0]], add=True)

        plsc.subcore_barrier()

        @pl.loop(0, NT_T // K)
        def _(i):
            pltpu.sync_copy(aggs.at[pl.ds(sid * NT_T + i * K, K)],
                            out_h.at[cid, pl.ds(sid * NT_T + i * K, K)])

    return pl.kernel(
        body,
        out_type=_sds(NC, NPAD, wm),
        mesh=_mesh(),
        scratch_types=(pltpu.VMEM((1, K), jnp.int32),
                       pltpu.VMEM((K, wm), F32),
                       pltpu.VMEM((K, wm), F32),
                       pltpu.VMEM_SHARED((NPAD, wm), F32),
                       pltpu.SemaphoreType.DMA),
    )(dst, wmsg)


def _tc_node1(x, w1b, w2, b2):
    tn = 1000

    def body(h_r, w1b_r, w2_r, b2_r, q_o, xr_o):
        hb = h_r[...]
        q_o[...] = jnp.dot(hb, w1b_r[...], preferred_element_type=F32, precision=HI)
        xr_o[...] = jnp.maximum(
            jnp.dot(hb, w2_r[...], preferred_element_type=F32, precision=HI) + b2_r[...], 0.0)

    full = lambda s: pl.BlockSpec(s, lambda i: (0,) * len(s))
    return pl.pallas_call(
        body,
        grid=(N // tn,),
        in_specs=[pl.BlockSpec((tn, W), lambda i: (i, 0)),
                  full((W, W)), full((W, W)), full((1, W))],
        out_specs=[pl.BlockSpec((tn, W), lambda i: (i, 0)),
                   pl.BlockSpec((tn, W), lambda i: (i, 0))],
        out_shape=(_sds(N, W), _sds(N, W)),
    )(x, w1b, w2, b2)


def _tc_node2(h, w1d, b1, w5, b5, w7t, w1b, w2, b2):
    tn = 1000

    def body(h_r, w1d_r, b1_r, w5_r, b5_r, w7t_r, w1b_r, w2_r, b2_r,
             td_o, q_o, xr_o):
        hb = h_r[...]
        p = jnp.dot(hb, w1d_r[...], preferred_element_type=F32, precision=HI) + b1_r[...]
        v = jnp.tanh(jnp.dot(hb, w5_r[...], preferred_element_type=F32, precision=HI)
                     + b5_r[...])
        td_o[...] = p + jnp.dot(v, w7t_r[...], preferred_element_type=F32, precision=HI)
        q_o[...] = jnp.dot(hb, w1b_r[...], preferred_element_type=F32, precision=HI)
        xr_o[...] = jnp.maximum(
            jnp.dot(hb, w2_r[...], preferred_element_type=F32, precision=HI) + b2_r[...], 0.0)

    full = lambda s: pl.BlockSpec(s, lambda i: (0,) * len(s))
    return pl.pallas_call(
        body,
        grid=(N // tn,),
        in_specs=[pl.BlockSpec((tn, W), lambda i: (i, 0)),
                  full((W, W)), full((1, W)),
                  full((W, 16)), full((1, 16)), full((16, W)),
                  full((W, W)), full((W, W)), full((1, W))],
        out_specs=[pl.BlockSpec((tn, W), lambda i: (i, 0)),
                   pl.BlockSpec((tn, W), lambda i: (i, 0)),
                   pl.BlockSpec((tn, W), lambda i: (i, 0))],
        out_shape=(_sds(N, W), _sds(N, W), _sds(N, W)),
    )(h, w1d, b1, w5, b5, w7t, w1b, w2, b2)


def _tc_edge1(xd, qs, w1d, b1, w5, b5, w7, w6, b6):
    te = 1000

    def body(xd_r, qs_r, w1d_r, b1_r, w5_r, b5_r, w7_r, w6_r, b6_r,
             msg_o, e_o):
        xdb = xd_r[...]
        msg = jnp.maximum(
            jnp.dot(xdb, w1d_r[...], preferred_element_type=F32, precision=HI) + b1_r[...]
            + qs_r[...], 0.0)
        msg_o[...] = msg
        vd = jnp.tanh(jnp.dot(xdb, w5_r[...], preferred_element_type=F32, precision=HI)
                      + b5_r[...]) * w7_r[...]
        t2 = jnp.tanh(jnp.dot(msg, w6_r[...], preferred_element_type=F32, precision=HI)
                      + b6_r[...])
        e_o[...] = jnp.exp(jnp.sum(vd * t2, axis=1, keepdims=True))

    full = lambda s: pl.BlockSpec(s, lambda i: (0,) * len(s))
    return pl.pallas_call(
        body,
        grid=(E // te,),
        in_specs=[pl.BlockSpec((te, W), lambda i: (i, 0)),
                  pl.BlockSpec((te, W), lambda i: (i, 0)),
                  full((W, W)), full((1, W)),
                  full((W, 16)), full((1, 16)), full((1, 16)),
                  full((W, 16)), full((1, 16))],
        out_specs=[pl.BlockSpec((te, W), lambda i: (i, 0)),
                   pl.BlockSpec((te, 1), lambda i: (i, 0))],
        out_shape=(_sds(E, W), _sds(E, 1)),
    )(xd, qs, w1d, b1, w5, b5, w7, w6, b6)


def _tc_edge2(td, qs, selv, w6, b6):
    te = 1000

    def body(td_r, qs_r, selv_r, w6_r, b6_r, msg_o, e_o):
        tdb = td_r[...]
        msg = jnp.maximum(tdb + qs_r[...], 0.0)
        msg_o[...] = msg
        vd = jnp.dot(tdb, selv_r[...], preferred_element_type=F32, precision=HI)
        t2 = jnp.tanh(jnp.dot(msg, w6_r[...], preferred_element_type=F32, precision=HI)
                      + b6_r[...])
        e_o[...] = jnp.exp(jnp.sum(vd * t2, axis=1, keepdims=True))

    full = lambda s: pl.BlockSpec(s, lambda i: (0,) * len(s))
    return pl.pallas_call(
        body,
        grid=(E // te,),
        in_specs=[pl.BlockSpec((te, W), lambda i: (i, 0)),
                  pl.BlockSpec((te, W), lambda i: (i, 0)),
                  full((W, 16)), full((W, 16)), full((1, 16))],
        out_specs=[pl.BlockSpec((te, W), lambda i: (i, 0)),
                   pl.BlockSpec((te, 1), lambda i: (i, 0))],
        out_shape=(_sds(E, W), _sds(E, 1)),
    )(td, qs, selv, w6, b6)


def _tc_reduce(partials3):
    def body(p_r, o_r):
        s = jnp.sum(p_r[...], axis=0)
        o_r[...] = 1.0 / (s + 1e-16)

    return pl.pallas_call(
        body,
        grid=(1,),
        in_specs=[pl.BlockSpec((NW, 8, NPAD // 8), lambda i: (0, 0, 0))],
        out_specs=pl.BlockSpec((8, NPAD // 8), lambda i: (0, 0)),
        out_shape=_sds(8, NPAD // 8),
    )(partials3)


def _tc_scale(msg, coef2):
    te = 1000
    wm = msg.shape[1]

    def body(m_r, c_r, o_r):
        o_r[...] = m_r[...] * c_r[...]

    return pl.pallas_call(
        body,
        grid=(E // te,),
        in_specs=[pl.BlockSpec((te, wm), lambda i: (i, 0)),
                  pl.BlockSpec((te, 1), lambda i: (i, 0))],
        out_specs=pl.BlockSpec((te, wm), lambda i: (i, 0)),
        out_shape=_sds(E, wm),
    )(msg, coef2)


def _tc_epilogue(aggp, xr, w3x, w3a, b3, w4x, w4a, b4):
    tn = 1000
    wm = aggp.shape[2]

    def body(ag_r, xr_r, w3x_r, w3a_r, b3_r, w4x_r, w4a_r, b4_r, o_r):
        agg = ag_r[0] + ag_r[1]
        xr_b = xr_r[...]
        z3 = (jnp.sum(xr_b * w3x_r[...], axis=1, keepdims=True)
              + jnp.sum(agg * w3a_r[...], axis=1, keepdims=True) + b3_r[0, 0])
        z4 = (jnp.sum(xr_b * w4x_r[...], axis=1, keepdims=True)
              + jnp.sum(agg * w4a_r[...], axis=1, keepdims=True) + b4_r[0, 0])
        a1 = 1.0 / (1.0 + jnp.exp(-z3))
        a2 = 1.0 / (1.0 + jnp.exp(-z4))
        hp = jnp.maximum(a1 * agg + a2 * xr_b[:, :wm], 0.0)
        if wm == W:
            o_r[...] = hp
        else:
            o_r[...] = jnp.concatenate(
                [hp, jnp.zeros((tn, W - wm), F32)], axis=1)

    full = lambda s: pl.BlockSpec(s, lambda i: (0,) * len(s))
    return pl.pallas_call(
        body,
        grid=(N // tn,),
        in_specs=[pl.BlockSpec((NC, tn, wm), lambda i: (0, i, 0)),
                  pl.BlockSpec((tn, W), lambda i: (i, 0)),
                  full((1, W)), full((1, wm)), full((1, 1)),
                  full((1, W)), full((1, wm)), full((1, 1))],
        out_specs=pl.BlockSpec((tn, W), lambda i: (i, 0)),
        out_shape=_sds(N, W),
    )(aggp, xr, w3x, w3a, b3, w4x, w4a, b4)


def _tc_pool(h, batch2, wgt, bg):
    def body(h_r, b_r, wg_r, bg_r, o_r):
        hb = h_r[...]
        gate = jnp.sum(hb * wg_r[...], axis=1, keepdims=True) + bg_r[0, 0]
        gids = lax.broadcasted_iota(jnp.int32, (1, G), 1)
        onehot = b_r[...] == gids
        mg = jnp.max(jnp.where(onehot, gate, -1e30), axis=0, keepdims=True)
        e2 = jnp.where(onehot, jnp.exp(gate - mg), 0.0)
        s = jnp.sum(e2, axis=0, keepdims=True)
        c2 = e2 / (s + 1e-16)
        o_r[...] = lax.dot_general(c2, hb, (((0,), (0,)), ((), ())),
                                   preferred_element_type=F32, precision=HI)

    return pl.pallas_call(
        body,
        grid=(1,),
        in_specs=[pl.BlockSpec((N, W), lambda i: (0, 0)),
                  pl.BlockSpec((N, 1), lambda i: (0, 0)),
                  pl.BlockSpec((1, W), lambda i: (0, 0)),
                  pl.BlockSpec((1, 1), lambda i: (0, 0))],
        out_specs=pl.BlockSpec((G, W), lambda i: (0, 0)),
        out_shape=_sds(G, W),
    )(h, batch2, wgt, bg)


def _tc_head(x1, x2, w1x, w1y, l1b, l2w, l2b, l3w, l3b, l4wt, l4b):
    def body(x1_r, x2_r, w1x_r, w1y_r, b1_r, w2_r, b2_r, w3_r, b3_r,
             w4_r, b4_r, o_r):
        h1 = jnp.maximum(
            jnp.dot(x1_r[...], w1x_r[...], preferred_element_type=F32, precision=HI)
            + jnp.dot(x2_r[...], w1y_r[...], preferred_element_type=F32, precision=HI)
            + b1_r[...], 0.0)
        h2 = jnp.maximum(jnp.dot(h1, w2_r[...], preferred_element_type=F32, precision=HI)
                         + b2_r[...], 0.0)
        h3 = jnp.dot(h2, w3_r[...], preferred_element_type=F32, precision=HI) + b3_r[...]
        o_r[...] = jnp.sum(h3 * w4_r[...], axis=1, keepdims=True) + b4_r[0, 0]

    full = lambda s: pl.BlockSpec(s, lambda i: (0,) * len(s))
    return pl.pallas_call(
        body,
        grid=(1,),
        in_specs=[full((G, W)), full((G, W)),
                  full((W, W)), full((W, W)), full((1, W)),
                  full((W, 16)), full((1, 16)),
                  full((16, 16)), full((1, 16)),
                  full((1, 16)), full((1, 1))],
        out_specs=full((G, 1)),
        out_shape=_sds(G, 1),
    )(x1, x2, w1x, w1y, l1b, l2w, l2b, l3w, l3b, l4wt, l4b)


def _pad2(w, rows, cols):
    return jnp.pad(w, ((0, rows - w.shape[0]), (0, cols - w.shape[1])))


def _pad1(b, n):
    return jnp.pad(b, (0, n - b.shape[0]))


def _conv1(x, dst, src, p):
    fin = fout = 128
    w1 = p["mlp1"]["w"]
    w1d = w1[:fin] - w1[fin:]
    q, xr = _tc_node1(x, w1[fin:], p["mlp2"]["w"], p["mlp2"]["b"][None, :])
    xd, qs = _sc_gather(dst, src, x, q)
    msg, e2 = _tc_edge1(xd, qs, w1d, p["mlp1"]["b"][None, :],
                        p["mlp5"]["w"], p["mlp5"]["b"][None, :],
                        p["mlp7"]["w"][:, 0][None, :],
                        p["mlp6"]["w"], p["mlp6"]["b"][None, :])
    return _edge_tail(dst, src, msg, e2, xr, p, fout)


def _conv2(h, dst, src, p, fin):
    fout = 64
    w1 = p["mlp1"]["w"]
    w1a, w1b = w1[:fin], w1[fin:]
    w1d_p = _pad2(w1a - w1b, W, W)
    w1b_p = _pad2(w1b, W, W)
    b1_p = _pad1(p["mlp1"]["b"], W)[None, :]
    w5_p = _pad2(p["mlp5"]["w"], W, 16)
    w7t = jnp.zeros((16, W), F32).at[
        jnp.arange(16), fout + jnp.arange(16)].set(p["mlp7"]["w"][:, 0])
    w2_p = _pad2(p["mlp2"]["w"], W, W)
    b2_p = _pad1(p["mlp2"]["b"], W)[None, :]
    td, q, xr = _tc_node2(h, w1d_p, b1_p, w5_p, p["mlp5"]["b"][None, :],
                          w7t, w1b_p, w2_p, b2_p)
    tdg, qsg = _sc_gather(dst, src, td, q)
    selv = jnp.zeros((W, 16), F32).at[
        fout + jnp.arange(16), jnp.arange(16)].set(1.0)
    w6_p = _pad2(p["mlp6"]["w"], W, 16)
    msg, e2 = _tc_edge2(tdg, qsg, selv, w6_p, p["mlp6"]["b"][None, :])
    return _edge_tail(dst, src, msg, e2, xr, p, fout)


def _edge_tail(dst, src, msg, e2, xr, p, fout):
    e1 = e2.reshape(E)
    partials = _sc_ssum(src, e1)
    sinv = _tc_reduce(partials.reshape(NW, 8, NPAD // 8)).reshape(NPAD)
    coef = _sc_coef(src, e1, sinv)
    wmsg = _tc_scale(msg, coef.reshape(E, 1))
    aggp = _sc_agg(dst, wmsg)
    w3 = p["mlp3"]["w"][:, 0]
    w4 = p["mlp4"]["w"][:, 0]
    return _tc_epilogue(aggp, xr,
                        _pad1(w3[:fout], W)[None, :],
                        _pad1(w3[fout:], W)[None, :],
                        p["mlp3"]["b"][None, None, 0],
                        _pad1(w4[:fout], W)[None, :],
                        _pad1(w4[fout:], W)[None, :],
                        p["mlp4"]["b"][None, None, 0])


def kernel(x, edge_index, batch, params):
    assert x.shape == (N, 128) and edge_index.shape == (2, E)
    src = edge_index[0]
    dst = edge_index[1]
    batch2 = batch.reshape(N, 1)

    h = _conv1(x, dst, src, params["conv1"])
    h = _conv2(h, dst, src, params["conv2"], 128)
    g1 = params["gate1"]
    x1 = _tc_pool(h, batch2, _pad1(g1["w"][:, 0], W)[None, :],
                  g1["b"][None, None, 0])
    h = _conv2(h, dst, src, params["conv3"], 64)
    h = _conv2(h, dst, src, params["conv4"], 64)
    g2 = params["gate2"]
    x2 = _tc_pool(h, batch2, _pad1(g2["w"][:, 0], W)[None, :],
                  g2["b"][None, None, 0])

    pr = params
    w1x = _pad2(pr["lin1"]["w"][:64], W, W)
    w1y = _pad2(pr["lin1"]["w"][64:], W, W)
    out = _tc_head(
        x1, x2, w1x, w1y, pr["lin1"]["b"][None, :],
        pr["lin2"]["w"], pr["lin2"]["b"][None, :],
        pr["lin3"]["w"], pr["lin3"]["b"][None, :],
        pr["lin4"]["w"][:, 0][None, :], pr["lin4"]["b"][None, None, 0])
    return out[:, 0]


# speedup vs baseline: 3.7057x; 1.0497x over previous
import jax
import jax.numpy as jnp
from jax import lax

HI = lax.Precision.HIGHEST
from jax.experimental import pallas as pl
from jax.experimental.pallas import tpu as pltpu
from jax.experimental.pallas import tpu_sc as plsc

F32 = jnp.float32

N = 10000
E = 320000
G = 16
W = 128
NPAD = 10240
NC, NS = 2, 16
NW = NC * NS
EW = E // NW
K = 80
CH = EW // K
NT_T = NPAD // NS


def _mesh():
    return plsc.VectorSubcoreMesh(core_axis_name="c", subcore_axis_name="s")


def _wid():
    return lax.axis_index("s") * NC + lax.axis_index("c")


def _sds(*shape):
    return jax.ShapeDtypeStruct(shape, F32)


def _sc_gather(dst, src, td, q):
    wt = td.shape[1]
    wq = q.shape[1]

    def body(dst_h, src_h, td_h, q_h, tdg_o, qsg_o, dsti, srci, tdv, qv,
             sem0, sem1):
        base = _wid() * EW

        def run(off, b, sem):
            pltpu.sync_copy(dst_h.at[pl.ds(off, K)], dsti.at[b])
            pltpu.sync_copy(src_h.at[pl.ds(off, K)], srci.at[b])
            c1 = pltpu.async_copy(td_h.at[dsti.at[b]], tdv.at[b], sem)
            c2 = pltpu.async_copy(q_h.at[srci.at[b]], qv.at[b], sem)
            return c1, c2

        def drain(off, b, c1, c2):
            c1.wait()
            c2.wait()
            pltpu.sync_copy(tdv.at[b], tdg_o.at[pl.ds(off, K)])
            pltpu.sync_copy(qv.at[b], qsg_o.at[pl.ds(off, K)])

        @pl.loop(0, CH // 2)
        def _(j):
            o0 = base + (2 * j) * K
            o1 = o0 + K
            c1, c2 = run(o0, 0, sem0)
            c3, c4 = run(o1, 1, sem1)
            drain(o0, 0, c1, c2)
            drain(o1, 1, c3, c4)

        if CH % 2:
            ot = base + (CH - 1) * K
            c1, c2 = run(ot, 0, sem0)
            drain(ot, 0, c1, c2)

    return pl.kernel(
        body,
        out_type=(_sds(E, wt), _sds(E, wq)),
        mesh=_mesh(),
        scratch_types=(pltpu.VMEM((2, K), jnp.int32),
                       pltpu.VMEM((2, K), jnp.int32),
                       pltpu.VMEM((2, K, wt), F32),
                       pltpu.VMEM((2, K, wq), F32),
                       pltpu.SemaphoreType.DMA,
                       pltpu.SemaphoreType.DMA),
    )(dst, src, td, q)


def _sc_ssum(src, e):
    def body(src_h, e_h, out_h, sidx, ev, acc, sem):
        wid = _wid()
        lane = lax.iota(jnp.int32, 16)
        rowv = lax.rem(lane, 8)
        mlow = lane < 8
        mhigh = lane >= 8

        @pl.loop(0, NPAD // 16)
        def _(i):
            for r in range(8):
                acc[r, pl.ds(i * 16, 16)] = jnp.zeros((16,), F32)

        @pl.loop(0, CH)
        def _(j):
            off = wid * EW + j * K
            pltpu.sync_copy(src_h.at[pl.ds(off, K)], sidx)
            pltpu.sync_copy(e_h.at[pl.ds(off, K)], ev)

            @pl.loop(0, K // 16)
            def _(g):
                sl = pl.ds(g * 16, 16)
                iv = sidx[sl]
                xv = ev[sl]
                plsc.addupdate_scatter(acc, [rowv, iv], xv, mask=mlow)
                plsc.addupdate_scatter(acc, [rowv, iv], xv, mask=mhigh)

        @pl.loop(0, NPAD // 16)
        def _(i):
            sl = pl.ds(i * 16, 16)
            s = acc[0, sl]
            for r in range(1, 8):
                s = s + acc[r, sl]
            acc[0, sl] = s

        pltpu.sync_copy(acc.at[0], out_h.at[wid])

    return pl.kernel(
        body,
        out_type=_sds(NW, NPAD),
        mesh=_mesh(),
        scratch_types=(pltpu.VMEM((K,), jnp.int32),
                       pltpu.VMEM((K,), F32),
                       pltpu.VMEM((8, NPAD), F32),
                       pltpu.SemaphoreType.DMA),
        compiler_params=pltpu.CompilerParams(needs_layout_passes=False),
    )(src, e)


def _sc_coef(src, e, sinv):
    def body(src_h, e_h, s_h, cf_o, sidx, ev, cf, sbuf, sem):
        wid = _wid()
        pltpu.sync_copy(s_h, sbuf)

        @pl.loop(0, CH)
        def _(j):
            off = wid * EW + j * K
            pltpu.sync_copy(src_h.at[pl.ds(off, K)], sidx)
            pltpu.sync_copy(e_h.at[pl.ds(off, K)], ev)

            @pl.loop(0, K // 16)
            def _(g):
                sl = pl.ds(g * 16, 16)
                cf[sl] = ev[sl] * plsc.load_gather(sbuf, [sidx[sl]])

            pltpu.sync_copy(cf, cf_o.at[pl.ds(off, K)])

    return pl.kernel(
        body,
        out_type=_sds(E),
        mesh=_mesh(),
        scratch_types=(pltpu.VMEM((K,), jnp.int32),
                       pltpu.VMEM((K,), F32),
                       pltpu.VMEM((K,), F32),
                       pltpu.VMEM((NPAD,), F32),
                       pltpu.SemaphoreType.DMA),
        compiler_params=pltpu.CompilerParams(needs_layout_passes=False),
    )(src, e, sinv)


def _sc_agg(dst, wmsg):
    wm = wmsg.shape[1]

    def body(dst_h, wm_h, out_h, idx2, wv, zv, aggs, sem0, sem1):
        cid = lax.axis_index("c")
        sid = lax.axis_index("s")
        wid = sid * NC + cid

        @pl.loop(0, K)
        def _(r):
            @pl.loop(0, wm // 16)
            def _(cc):
                zv[r, pl.ds(cc * 16, 16)] = jnp.zeros((16,), F32)

        @pl.loop(0, NT_T // K)
        def _(i):
            pltpu.sync_copy(zv, aggs.at[pl.ds(sid * NT_T + i * K, K)])

        plsc.subcore_barrier()

        def load(off, b, sem):
            c1 = pltpu.async_copy(dst_h.at[pl.ds(off, K)], idx2.at[b], sem)
            c2 = pltpu.async_copy(wm_h.at[pl.ds(off, K)], wv.at[b], sem)
            return c1, c2

        def scat(b, c1, c2):
            c1.wait()
            c2.wait()
            pltpu.sync_copy(wv.at[b], aggs.at[idx2.at[b]], add=True)

        @pl.loop(0, CH // 2)
        def _(j):
            o0 = wid * EW + (2 * j) * K
            o1 = o0 + K
            c1, c2 = load(o0, 0, sem0)
            c3, c4 = load(o1, 1, sem1)
            scat(0, c1, c2)
            scat(1, c3, c4)

        if CH % 2:
            ot = wid * EW + (CH - 1) * K
            c1, c2 = load(ot, 0, sem0)
            scat(0, c1, c2)

        plsc.subcore_barrier()

        @pl.loop(0, NT_T // K)
        def _(i):
            pltpu.sync_copy(aggs.at[pl.ds(sid * NT_T + i * K, K)],
                            out_h.at[cid, pl.ds(sid * NT_T + i * K, K)])

    return pl.kernel(
        body,
        out_type=_sds(NC, NPAD, wm),
        mesh=_mesh(),
        scratch_types=(pltpu.VMEM((2, K), jnp.int32),
                       pltpu.VMEM((2, K, wm), F32),
                       pltpu.VMEM((K, wm), F32),
                       pltpu.VMEM_SHARED((NPAD, wm), F32),
                       pltpu.SemaphoreType.DMA,
                       pltpu.SemaphoreType.DMA),
    )(dst, wmsg)


def _tc_node1(x, w1b, w2, b2):
    tn = 1000

    def body(h_r, w1b_r, w2_r, b2_r, q_o, xr_o):
        hb = h_r[...]
        q_o[...] = jnp.dot(hb, w1b_r[...], preferred_element_type=F32, precision=HI)
        xr_o[...] = jnp.maximum(
            jnp.dot(hb, w2_r[...], preferred_element_type=F32, precision=HI) + b2_r[...], 0.0)

    full = lambda s: pl.BlockSpec(s, lambda i: (0,) * len(s))
    return pl.pallas_call(
        body,
        grid=(N // tn,),
        in_specs=[pl.BlockSpec((tn, W), lambda i: (i, 0)),
                  full((W, W)), full((W, W)), full((1, W))],
        out_specs=[pl.BlockSpec((tn, W), lambda i: (i, 0)),
                   pl.BlockSpec((tn, W), lambda i: (i, 0))],
        out_shape=(_sds(N, W), _sds(N, W)),
    )(x, w1b, w2, b2)


def _tc_node2(h, w1d, b1, w5, b5, w7t, w1b, w2, b2):
    tn = 1000

    def body(h_r, w1d_r, b1_r, w5_r, b5_r, w7t_r, w1b_r, w2_r, b2_r,
             td_o, q_o, xr_o):
        hb = h_r[...]
        p = jnp.dot(hb, w1d_r[...], preferred_element_type=F32, precision=HI) + b1_r[...]
        v = jnp.tanh(jnp.dot(hb, w5_r[...], preferred_element_type=F32, precision=HI)
                     + b5_r[...])
        td_o[...] = p + jnp.dot(v, w7t_r[...], preferred_element_type=F32, precision=HI)
        q_o[...] = jnp.dot(hb, w1b_r[...], preferred_element_type=F32, precision=HI)
        xr_o[...] = jnp.maximum(
            jnp.dot(hb, w2_r[...], preferred_element_type=F32, precision=HI) + b2_r[...], 0.0)

    full = lambda s: pl.BlockSpec(s, lambda i: (0,) * len(s))
    return pl.pallas_call(
        body,
        grid=(N // tn,),
        in_specs=[pl.BlockSpec((tn, W), lambda i: (i, 0)),
                  full((W, W)), full((1, W)),
                  full((W, 16)), full((1, 16)), full((16, W)),
                  full((W, W)), full((W, W)), full((1, W))],
        out_specs=[pl.BlockSpec((tn, W), lambda i: (i, 0)),
                   pl.BlockSpec((tn, W), lambda i: (i, 0)),
                   pl.BlockSpec((tn, W), lambda i: (i, 0))],
        out_shape=(_sds(N, W), _sds(N, W), _sds(N, W)),
    )(h, w1d, b1, w5, b5, w7t, w1b, w2, b2)


def _tc_edge1(xd, qs, w1d, b1, w5, b5, w7, w6, b6):
    te = 1000

    def body(xd_r, qs_r, w1d_r, b1_r, w5_r, b5_r, w7_r, w6_r, b6_r,
             msg_o, e_o):
        xdb = xd_r[...]
        msg = jnp.maximum(
            jnp.dot(xdb, w1d_r[...], preferred_element_type=F32, precision=HI) + b1_r[...]
            + qs_r[...], 0.0)
        msg_o[...] = msg
        vd = jnp.tanh(jnp.dot(xdb, w5_r[...], preferred_element_type=F32, precision=HI)
                      + b5_r[...]) * w7_r[...]
        t2 = jnp.tanh(jnp.dot(msg, w6_r[...], preferred_element_type=F32, precision=HI)
                      + b6_r[...])
        e_o[...] = jnp.exp(jnp.sum(vd * t2, axis=1, keepdims=True))

    full = lambda s: pl.BlockSpec(s, lambda i: (0,) * len(s))
    return pl.pallas_call(
        body,
        grid=(E // te,),
        in_specs=[pl.BlockSpec((te, W), lambda i: (i, 0)),
                  pl.BlockSpec((te, W), lambda i: (i, 0)),
                  full((W, W)), full((1, W)),
                  full((W, 16)), full((1, 16)), full((1, 16)),
                  full((W, 16)), full((1, 16))],
        out_specs=[pl.BlockSpec((te, W), lambda i: (i, 0)),
                   pl.BlockSpec((te, 1), lambda i: (i, 0))],
        out_shape=(_sds(E, W), _sds(E, 1)),
    )(xd, qs, w1d, b1, w5, b5, w7, w6, b6)


def _tc_edge2(td, qs, selv, w6, b6):
    te = 1000

    def body(td_r, qs_r, selv_r, w6_r, b6_r, msg_o, e_o):
        tdb = td_r[...]
        msg = jnp.maximum(tdb + qs_r[...], 0.0)
        msg_o[...] = msg
        vd = jnp.dot(tdb, selv_r[...], preferred_element_type=F32, precision=HI)
        t2 = jnp.tanh(jnp.dot(msg, w6_r[...], preferred_element_type=F32, precision=HI)
                      + b6_r[...])
        e_o[...] = jnp.exp(jnp.sum(vd * t2, axis=1, keepdims=True))

    full = lambda s: pl.BlockSpec(s, lambda i: (0,) * len(s))
    return pl.pallas_call(
        body,
        grid=(E // te,),
        in_specs=[pl.BlockSpec((te, W), lambda i: (i, 0)),
                  pl.BlockSpec((te, W), lambda i: (i, 0)),
                  full((W, 16)), full((W, 16)), full((1, 16))],
        out_specs=[pl.BlockSpec((te, W), lambda i: (i, 0)),
                   pl.BlockSpec((te, 1), lambda i: (i, 0))],
        out_shape=(_sds(E, W), _sds(E, 1)),
    )(td, qs, selv, w6, b6)


def _tc_reduce(partials3):
    def body(p_r, o_r):
        s = jnp.sum(p_r[...], axis=0)
        o_r[...] = 1.0 / (s + 1e-16)

    return pl.pallas_call(
        body,
        grid=(1,),
        in_specs=[pl.BlockSpec((NW, 8, NPAD // 8), lambda i: (0, 0, 0))],
        out_specs=pl.BlockSpec((8, NPAD // 8), lambda i: (0, 0)),
        out_shape=_sds(8, NPAD // 8),
    )(partials3)


def _tc_scale(msg, coef2):
    te = 1000
    wm = msg.shape[1]

    def body(m_r, c_r, o_r):
        o_r[...] = m_r[...] * c_r[...]

    return pl.pallas_call(
        body,
        grid=(E // te,),
        in_specs=[pl.BlockSpec((te, wm), lambda i: (i, 0)),
                  pl.BlockSpec((te, 1), lambda i: (i, 0))],
        out_specs=pl.BlockSpec((te, wm), lambda i: (i, 0)),
        out_shape=_sds(E, wm),
    )(msg, coef2)


def _tc_epilogue(aggp, xr, w3x, w3a, b3, w4x, w4a, b4):
    tn = 1000
    wm = aggp.shape[2]

    def body(ag_r, xr_r, w3x_r, w3a_r, b3_r, w4x_r, w4a_r, b4_r, o_r):
        agg = ag_r[0] + ag_r[1]
        xr_b = xr_r[...]
        z3 = (jnp.sum(xr_b * w3x_r[...], axis=1, keepdims=True)
              + jnp.sum(agg * w3a_r[...], axis=1, keepdims=True) + b3_r[0, 0])
        z4 = (jnp.sum(xr_b * w4x_r[...], axis=1, keepdims=True)
              + jnp.sum(agg * w4a_r[...], axis=1, keepdims=True) + b4_r[0, 0])
        a1 = 1.0 / (1.0 + jnp.exp(-z3))
        a2 = 1.0 / (1.0 + jnp.exp(-z4))
        hp = jnp.maximum(a1 * agg + a2 * xr_b[:, :wm], 0.0)
        if wm == W:
            o_r[...] = hp
        else:
            o_r[...] = jnp.concatenate(
                [hp, jnp.zeros((tn, W - wm), F32)], axis=1)

    full = lambda s: pl.BlockSpec(s, lambda i: (0,) * len(s))
    return pl.pallas_call(
        body,
        grid=(N // tn,),
        in_specs=[pl.BlockSpec((NC, tn, wm), lambda i: (0, i, 0)),
                  pl.BlockSpec((tn, W), lambda i: (i, 0)),
                  full((1, W)), full((1, wm)), full((1, 1)),
                  full((1, W)), full((1, wm)), full((1, 1))],
        out_specs=pl.BlockSpec((tn, W), lambda i: (i, 0)),
        out_shape=_sds(N, W),
    )(aggp, xr, w3x, w3a, b3, w4x, w4a, b4)


def _tc_pool(h, batch2, wgt, bg):
    def body(h_r, b_r, wg_r, bg_r, o_r):
        hb = h_r[...]
        gate = jnp.sum(hb * wg_r[...], axis=1, keepdims=True) + bg_r[0, 0]
        gids = lax.broadcasted_iota(jnp.int32, (1, G), 1)
        onehot = b_r[...] == gids
        mg = jnp.max(jnp.where(onehot, gate, -1e30), axis=0, keepdims=True)
        e2 = jnp.where(onehot, jnp.exp(gate - mg), 0.0)
        s = jnp.sum(e2, axis=0, keepdims=True)
        c2 = e2 / (s + 1e-16)
        o_r[...] = lax.dot_general(c2, hb, (((0,), (0,)), ((), ())),
                                   preferred_element_type=F32, precision=HI)

    return pl.pallas_call(
        body,
        grid=(1,),
        in_specs=[pl.BlockSpec((N, W), lambda i: (0, 0)),
                  pl.BlockSpec((N, 1), lambda i: (0, 0)),
                  pl.BlockSpec((1, W), lambda i: (0, 0)),
                  pl.BlockSpec((1, 1), lambda i: (0, 0))],
        out_specs=pl.BlockSpec((G, W), lambda i: (0, 0)),
        out_shape=_sds(G, W),
    )(h, batch2, wgt, bg)


def _tc_head(x1, x2, w1x, w1y, l1b, l2w, l2b, l3w, l3b, l4wt, l4b):
    def body(x1_r, x2_r, w1x_r, w1y_r, b1_r, w2_r, b2_r, w3_r, b3_r,
             w4_r, b4_r, o_r):
        h1 = jnp.maximum(
            jnp.dot(x1_r[...], w1x_r[...], preferred_element_type=F32, precision=HI)
            + jnp.dot(x2_r[...], w1y_r[...], preferred_element_type=F32, precision=HI)
            + b1_r[...], 0.0)
        h2 = jnp.maximum(jnp.dot(h1, w2_r[...], preferred_element_type=F32, precision=HI)
                         + b2_r[...], 0.0)
        h3 = jnp.dot(h2, w3_r[...], preferred_element_type=F32, precision=HI) + b3_r[...]
        o_r[...] = jnp.sum(h3 * w4_r[...], axis=1, keepdims=True) + b4_r[0, 0]

    full = lambda s: pl.BlockSpec(s, lambda i: (0,) * len(s))
    return pl.pallas_call(
        body,
        grid=(1,),
        in_specs=[full((G, W)), full((G, W)),
                  full((W, W)), full((W, W)), full((1, W)),
                  full((W, 16)), full((1, 16)),
                  full((16, 16)), full((1, 16)),
                  full((1, 16)), full((1, 1))],
        out_specs=full((G, 1)),
        out_shape=_sds(G, 1),
    )(x1, x2, w1x, w1y, l1b, l2w, l2b, l3w, l3b, l4wt, l4b)


def _pad2(w, rows, cols):
    return jnp.pad(w, ((0, rows - w.shape[0]), (0, cols - w.shape[1])))


def _pad1(b, n):
    return jnp.pad(b, (0, n - b.shape[0]))


def _conv1(x, dst, src, p):
    fin = fout = 128
    w1 = p["mlp1"]["w"]
    w1d = w1[:fin] - w1[fin:]
    q, xr = _tc_node1(x, w1[fin:], p["mlp2"]["w"], p["mlp2"]["b"][None, :])
    xd, qs = _sc_gather(dst, src, x, q)
    msg, e2 = _tc_edge1(xd, qs, w1d, p["mlp1"]["b"][None, :],
                        p["mlp5"]["w"], p["mlp5"]["b"][None, :],
                        p["mlp7"]["w"][:, 0][None, :],
                        p["mlp6"]["w"], p["mlp6"]["b"][None, :])
    return _edge_tail(dst, src, msg, e2, xr, p, fout)


def _conv2(h, dst, src, p, fin):
    fout = 64
    w1 = p["mlp1"]["w"]
    w1a, w1b = w1[:fin], w1[fin:]
    w1d_p = _pad2(w1a - w1b, W, W)
    w1b_p = _pad2(w1b, W, W)
    b1_p = _pad1(p["mlp1"]["b"], W)[None, :]
    w5_p = _pad2(p["mlp5"]["w"], W, 16)
    w7t = jnp.zeros((16, W), F32).at[
        jnp.arange(16), fout + jnp.arange(16)].set(p["mlp7"]["w"][:, 0])
    w2_p = _pad2(p["mlp2"]["w"], W, W)
    b2_p = _pad1(p["mlp2"]["b"], W)[None, :]
    td, q, xr = _tc_node2(h, w1d_p, b1_p, w5_p, p["mlp5"]["b"][None, :],
                          w7t, w1b_p, w2_p, b2_p)
    tdg, qsg = _sc_gather(dst, src, td, q)
    selv = jnp.zeros((W, 16), F32).at[
        fout + jnp.arange(16), jnp.arange(16)].set(1.0)
    w6_p = _pad2(p["mlp6"]["w"], W, 16)
    msg, e2 = _tc_edge2(tdg, qsg, selv, w6_p, p["mlp6"]["b"][None, :])
    return _edge_tail(dst, src, msg, e2, xr, p, fout)


def _edge_tail(dst, src, msg, e2, xr, p, fout):
    e1 = e2.reshape(E)
    partials = _sc_ssum(src, e1)
    sinv = _tc_reduce(partials.reshape(NW, 8, NPAD // 8)).reshape(NPAD)
    coef = _sc_coef(src, e1, sinv)
    wmsg = _tc_scale(msg, coef.reshape(E, 1))
    aggp = _sc_agg(dst, wmsg)
    w3 = p["mlp3"]["w"][:, 0]
    w4 = p["mlp4"]["w"][:, 0]
    return _tc_epilogue(aggp, xr,
                        _pad1(w3[:fout], W)[None, :],
                        _pad1(w3[fout:], W)[None, :],
                        p["mlp3"]["b"][None, None, 0],
                        _pad1(w4[:fout], W)[None, :],
                        _pad1(w4[fout:], W)[None, :],
                        p["mlp4"]["b"][None, None, 0])


def kernel(x, edge_index, batch, params):
    assert x.shape == (N, 128) and edge_index.shape == (2, E)
    src = edge_index[0]
    dst = edge_index[1]
    batch2 = batch.reshape(N, 1)

    h = _conv1(x, dst, src, params["conv1"])
    h = _conv2(h, dst, src, params["conv2"], 128)
    g1 = params["gate1"]
    x1 = _tc_pool(h, batch2, _pad1(g1["w"][:, 0], W)[None, :],
                  g1["b"][None, None, 0])
    h = _conv2(h, dst, src, params["conv3"], 64)
    h = _conv2(h, dst, src, params["conv4"], 64)
    g2 = params["gate2"]
    x2 = _tc_pool(h, batch2, _pad1(g2["w"][:, 0], W)[None, :],
                  g2["b"][None, None, 0])

    pr = params
    w1x = _pad2(pr["lin1"]["w"][:64], W, W)
    w1y = _pad2(pr["lin1"]["w"][64:], W, W)
    out = _tc_head(
        x1, x2, w1x, w1y, pr["lin1"]["b"][None, :],
        pr["lin2"]["w"], pr["lin2"]["b"][None, :],
        pr["lin3"]["w"], pr["lin3"]["b"][None, :],
        pr["lin4"]["w"][:, 0][None, :], pr["lin4"]["b"][None, None, 0])
    return out[:, 0]
